# Initial kernel scaffold; baseline (speedup 1.0000x reference)
import functools

import jax
import jax.numpy as jnp
from jax import lax
from jax.experimental import pallas as pl
from jax.experimental.pallas import tpu as pltpu
from jax.experimental.pallas import tpu_sc as plsc

NUM_WORKERS = 32
GROUP = 2048


@functools.lru_cache(maxsize=None)
def _make_gather(n: int, embed: int):
    assert n % (NUM_WORKERS * GROUP) == 0
    n_per_worker = n // NUM_WORKERS
    n_groups = n_per_worker // GROUP
    mesh = plsc.VectorSubcoreMesh(core_axis_name="c", subcore_axis_name="s")

    @functools.partial(
        pl.kernel,
        mesh=mesh,
        out_type=jax.ShapeDtypeStruct((n, embed), jnp.float32),
        scratch_types=[
            pltpu.VMEM((GROUP,), jnp.int32),
            pltpu.VMEM((GROUP, embed), jnp.float32),
            pltpu.SemaphoreType.DMA,
        ],
    )
    def gather_kernel(idx_hbm, table_hbm, out_hbm, idx_v, rows_v, sem):
        wid = lax.axis_index("s") * 2 + lax.axis_index("c")
        base0 = wid * n_per_worker

        def body(g, carry):
            base = base0 + g * GROUP
            pltpu.sync_copy(idx_hbm.at[pl.ds(base, GROUP)], idx_v)
            pltpu.async_copy(table_hbm.at[idx_v], rows_v, sem).wait()
            pltpu.sync_copy(rows_v, out_hbm.at[pl.ds(base, GROUP)])
            return carry

        lax.fori_loop(0, n_groups, body, 0)

    return gather_kernel


def kernel(input, table):
    batch, hist = input.shape
    vocab, embed = table.shape
    idx = input.reshape(-1).astype(jnp.int32)
    out = _make_gather(batch * hist, embed)(idx, table)
    return out.reshape(batch, hist, embed)


# speedup vs baseline: 4.9457x; 4.9457x over previous
import functools

import jax
import jax.numpy as jnp
from jax import lax
from jax.experimental import pallas as pl
from jax.experimental.pallas import tpu as pltpu
from jax.experimental.pallas import tpu_sc as plsc

NUM_WORKERS = 32
GROUP = 2048


@functools.lru_cache(maxsize=None)
def _make_gather(n: int, embed: int):
    assert n % (NUM_WORKERS * GROUP) == 0
    n_per_worker = n // NUM_WORKERS
    n_groups = n_per_worker // GROUP
    mesh = plsc.VectorSubcoreMesh(core_axis_name="c", subcore_axis_name="s")

    @functools.partial(
        pl.kernel,
        mesh=mesh,
        out_type=jax.ShapeDtypeStruct((n, embed), jnp.float32),
        scratch_types=[
            pltpu.VMEM((GROUP,), jnp.int32),
            pltpu.VMEM((GROUP, embed), jnp.float32),
            pltpu.SemaphoreType.DMA,
        ],
        compiler_params=pltpu.CompilerParams(use_tc_tiling_on_sc=False),
    )
    def gather_kernel(idx_hbm, table_hbm, out_hbm, idx_v, rows_v, sem):
        wid = lax.axis_index("s") * 2 + lax.axis_index("c")
        base0 = wid * n_per_worker

        def body(g, carry):
            base = base0 + g * GROUP
            pltpu.sync_copy(idx_hbm.at[pl.ds(base, GROUP)], idx_v)
            pltpu.async_copy(table_hbm.at[idx_v], rows_v, sem).wait()
            pltpu.sync_copy(rows_v, out_hbm.at[pl.ds(base, GROUP)])
            return carry

        lax.fori_loop(0, n_groups, body, 0)

    return gather_kernel


def kernel(input, table):
    batch, hist = input.shape
    vocab, embed = table.shape
    idx = input.reshape(-1).astype(jnp.int32)
    out = _make_gather(batch * hist, embed)(idx, table)
    return out.reshape(batch, hist, embed)


# speedup vs baseline: 5.0444x; 1.0199x over previous
import functools

import jax
import jax.numpy as jnp
from jax import lax
from jax.experimental import pallas as pl
from jax.experimental.pallas import tpu as pltpu
from jax.experimental.pallas import tpu_sc as plsc

NUM_WORKERS = 32
GROUP = 800
NBUF = 4


@functools.lru_cache(maxsize=None)
def _make_gather(n: int, embed: int):
    assert n % (NUM_WORKERS * NBUF * GROUP) == 0
    n_per_worker = n // NUM_WORKERS
    n_groups = n_per_worker // GROUP
    mesh = plsc.VectorSubcoreMesh(core_axis_name="c", subcore_axis_name="s")

    @functools.partial(
        pl.kernel,
        mesh=mesh,
        out_type=jax.ShapeDtypeStruct((n, embed), jnp.float32),
        scratch_types=[
            pltpu.VMEM((NBUF, GROUP), jnp.int32),
            pltpu.VMEM((NBUF, GROUP, embed), jnp.float32),
            [pltpu.SemaphoreType.DMA] * NBUF,
            [pltpu.SemaphoreType.DMA] * NBUF,
        ],
        compiler_params=pltpu.CompilerParams(use_tc_tiling_on_sc=False),
    )
    def gather_kernel(idx_hbm, table_hbm, out_hbm, idx_v, rows_v, gsems, ssems):
        wid = lax.axis_index("s") * 2 + lax.axis_index("c")
        base0 = wid * n_per_worker

        def load_and_gather(g, b):
            base = base0 + g * GROUP
            pltpu.sync_copy(idx_hbm.at[pl.ds(base, GROUP)], idx_v.at[b])
            pltpu.async_copy(table_hbm.at[idx_v.at[b]], rows_v.at[b], gsems[b])

        def wait_gather(b):
            pltpu.make_async_copy(
                table_hbm.at[idx_v.at[b]], rows_v.at[b], gsems[b]
            ).wait()

        def store(g, b):
            base = base0 + g * GROUP
            pltpu.async_copy(rows_v.at[b], out_hbm.at[pl.ds(base, GROUP)], ssems[b])

        def wait_store(g, b):
            base = base0 + g * GROUP
            pltpu.make_async_copy(
                rows_v.at[b], out_hbm.at[pl.ds(base, GROUP)], ssems[b]
            ).wait()

        for b in range(NBUF):
            load_and_gather(b, b)
        for b in range(NBUF):
            wait_gather(b)
            store(b, b)

        @pl.loop(NBUF, n_groups, step=NBUF)
        def _(gg):
            for b in range(NBUF):
                wait_store(gg + b - NBUF, b)
                load_and_gather(gg + b, b)
            for b in range(NBUF):
                wait_gather(b)
                store(gg + b, b)

        for b in range(NBUF):
            wait_store(n_groups + b - NBUF, b)

    return gather_kernel


def kernel(input, table):
    batch, hist = input.shape
    vocab, embed = table.shape
    idx = input.reshape(-1).astype(jnp.int32)
    out = _make_gather(batch * hist, embed)(idx, table)
    return out.reshape(batch, hist, embed)
